# baseline (speedup 1.0000x reference)
import jax
import jax.numpy as jnp
from jax import lax
from jax.experimental import pallas as pl
from jax.experimental.pallas import tpu as pltpu
from jax.experimental.pallas import tpu_sc as plsc

VOCAB = 1000000
D_MODEL = 64
SEQ = 200
BATCH = 4096

NC = 2
NS = 16
NW = NC * NS

NCT_FULL = VOCAB // 128
CT_TAIL = VOCAB - NCT_FULL * 128
PAIRS = VOCAB // 2
CT_PER_W = 244

UNITS = (SEQ // 8) * (BATCH // 128)
UNITS_PER_W = UNITS // NW


def _iota16():
    return lax.iota(jnp.int32, 16)


def _splat(x):
    return jnp.zeros((16,), jnp.int32) + x


def _repack_body(tab_hbm, tail_hbm, pack_hbm, in_v, tr_v, *sems):
    rs = sems[:4]
    ws = sems[4:]
    wid = lax.axis_index("s") * NC + lax.axis_index("c")
    start = wid * CT_PER_W
    rows4 = [h * 16 + _iota16() for h in range(4)]

    def read(ct, b):
        return pltpu.make_async_copy(
            tab_hbm.at[:, pl.ds(ct * 128, 128)], in_v.at[b], rs[b])

    def write(ct, b):
        return pltpu.make_async_copy(
            tr_v.at[b], pack_hbm.at[pl.ds(ct * 64, 64)], ws[b])

    def transpose(b, n_pairs):
        src = in_v.at[b]
        dst = tr_v.at[b]

        @plsc.parallel_loop(0, n_pairs, unroll=4)
        def _(p):
            col0 = _splat(2 * p)
            col1 = col0 + 1
            for h in range(8):
                col = col1 if h >= 4 else col0
                dst[p, pl.ds(h * 16, 16)] = plsc.load_gather(
                    src, [rows4[h % 4], col])

    for b in range(4):
        read(start + b, b).start()

    @pl.loop(0, CT_PER_W // 4)
    def _(g):
        for b in range(4):
            ct = start + 4 * g + b
            read(ct, b).wait()

            @pl.when(g > 0)
            def _():
                write(ct - 4, b).wait()

            transpose(b, 64)
            write(ct, b).start()

            @pl.when(g < CT_PER_W // 4 - 1)
            def _():
                read(ct + 4, b).start()

    for b in range(4):
        write(start + CT_PER_W - 4 + b, b).wait()

    @pl.when(wid < 4)
    def _():
        ct = NW * CT_PER_W + wid
        pltpu.sync_copy(tab_hbm.at[:, pl.ds(ct * 128, 128)], in_v.at[0])
        transpose(0, 64)
        write(ct, 0).start()
        write(ct, 0).wait()

    @pl.when(wid == 31)
    def _():
        pltpu.sync_copy(tail_hbm, tr_v.at[1, pl.ds(0, CT_TAIL // 2)])
        pltpu.make_async_copy(
            tr_v.at[1, pl.ds(0, CT_TAIL // 2)],
            pack_hbm.at[pl.ds(NCT_FULL * 64, CT_TAIL // 2)], ws[1]).start()
        pltpu.make_async_copy(
            tr_v.at[1, pl.ds(0, CT_TAIL // 2)],
            pack_hbm.at[pl.ds(NCT_FULL * 64, CT_TAIL // 2)], ws[1]).wait()


def _lookup_body(pack_hbm, tok_hbm, out_hbm, tok_v, idx_v, gath_v, outb_v,
                 *sems):
    gs = sems[:4]
    os = sems[4:]
    wid = lax.axis_index("s") * NC + lax.axis_index("c")
    rows8 = [h * 16 + _iota16() for h in range(8)]

    def gather(s_i, b):
        return pltpu.make_async_copy(
            pack_hbm.at[idx_v.at[s_i]], gath_v.at[b], gs[b])

    def out_write(s, bct, b):
        return pltpu.make_async_copy(
            outb_v.at[b], out_hbm.at[s, :, bct], os[b])

    @pl.loop(0, UNITS_PER_W)
    def _(uu):
        u = wid * UNITS_PER_W + uu
        s0 = (u // 32) * 8
        bct = u % 32
        pltpu.sync_copy(tok_hbm.at[pl.ds(s0, 8), pl.ds(bct * 128, 128)],
                        tok_v)
        for s_i in range(8):
            for h in range(8):
                t = tok_v[s_i, pl.ds(h * 16, 16)]
                idx_v[s_i, pl.ds(h * 16, 16)] = lax.shift_right_logical(t, 1)

        for s_i in range(3):
            gather(s_i, s_i).start()
        for s_i in range(8):
            b = s_i % 4
            gather(s_i, b).wait()
            if s_i < 5:
                gather(s_i + 3, (s_i + 3) % 4).start()
            if s_i < 4:
                @pl.when(uu > 0)
                def _():
                    out_write(0, 0, b).wait()
            else:
                out_write(0, 0, b).wait()
            src = gath_v.at[b]
            dst = outb_v.at[b]
            bases = []
            for h in range(8):
                par = lax.bitwise_and(tok_v[s_i, pl.ds(h * 16, 16)], 1)
                bases.append(par * 64)
            for fg in range(8):
                @plsc.parallel_loop(0, 8, unroll=2)
                def _(f8, fg=fg):
                    f = fg * 8 + f8
                    for h in range(8):
                        dst[fg, f8, pl.ds(h * 16, 16)] = plsc.load_gather(
                            src, [rows8[h], bases[h] + f])
            out_write(s0 + s_i, bct, b).start()

    for b in range(4):
        out_write(0, 0, b).wait()


@jax.jit
def kernel(tokens, emb_weight):
    tok_t = jnp.transpose(tokens).astype(jnp.int32)
    tab_t = jnp.transpose(emb_weight)
    tail2 = lax.slice(emb_weight, (NCT_FULL * 128, 0), (VOCAB, D_MODEL))
    tail2 = tail2.reshape(CT_TAIL // 2, 128)
    mesh = plsc.VectorSubcoreMesh(core_axis_name="c", subcore_axis_name="s")
    params = pltpu.CompilerParams(use_tc_tiling_on_sc=True,
                                  needs_layout_passes=False)

    pack = pl.kernel(
        _repack_body,
        out_type=jax.ShapeDtypeStruct((PAIRS, 128), jnp.float32),
        mesh=mesh,
        scratch_types=[
            pltpu.VMEM((4, 64, 128), jnp.float32),
            pltpu.VMEM((4, 64, 128), jnp.float32),
        ] + [pltpu.SemaphoreType.DMA] * 8,
        compiler_params=params,
    )(tab_t, tail2)

    out5 = pl.kernel(
        _lookup_body,
        out_type=jax.ShapeDtypeStruct((SEQ, 8, 32, 8, 128), jnp.float32),
        mesh=mesh,
        scratch_types=[
            pltpu.VMEM((8, 128), jnp.int32),
            pltpu.VMEM((8, 128), jnp.int32),
            pltpu.VMEM((4, 128, 128), jnp.float32),
            pltpu.VMEM((4, 8, 8, 128), jnp.float32),
        ] + [pltpu.SemaphoreType.DMA] * 8,
        compiler_params=params,
    )(pack, tok_t)

    out = jnp.transpose(out5, (2, 4, 0, 1, 3))
    return out.reshape(BATCH, SEQ, D_MODEL)


# speedup vs baseline: 2.1235x; 2.1235x over previous
import jax
import jax.numpy as jnp
from jax import lax
from jax.experimental import pallas as pl
from jax.experimental.pallas import tpu as pltpu
from jax.experimental.pallas import tpu_sc as plsc

VOCAB = 1000000
D_MODEL = 64
SEQ = 200
BATCH = 4096

NC = 2
NS = 16
NW = NC * NS

NCT_FULL = VOCAB // 128
CT_TAIL = VOCAB - NCT_FULL * 128
PAIRS = VOCAB // 2
CT_PER_W = 244

UNITS = (SEQ // 8) * (BATCH // 128)
UNITS_PER_W = UNITS // NW


def _iota16():
    return lax.iota(jnp.int32, 16)


def _splat(x):
    return jnp.zeros((16,), jnp.int32) + x


def _repack_body(tab_hbm, tail_hbm, pack_hbm, in_v, tr_v, spad, *sems):
    rs = sems[:4]
    ws = sems[4:]
    wid = lax.axis_index("s") * NC + lax.axis_index("c")
    start = wid * CT_PER_W
    vc65 = [(h * 16 + _iota16()) * 65 for h in range(8)]

    def read(ct, b):
        return pltpu.make_async_copy(
            tab_hbm.at[:, pl.ds(ct * 128, 128)], in_v.at[b], rs[b])

    def write(ct, b):
        return pltpu.make_async_copy(
            tr_v.at[b], pack_hbm.at[pl.ds(ct * 64, 64)], ws[b])

    def transpose(b, n_pairs):
        src = in_v.at[b]
        dst = tr_v.at[b]

        @plsc.parallel_loop(0, 64, unroll=2)
        def _(f):
            for h in range(8):
                plsc.store_scatter(spad, [vc65[h] + f],
                                   src[f, pl.ds(h * 16, 16)])

        @plsc.parallel_loop(0, n_pairs, unroll=2)
        def _(p):
            for h in range(8):
                base = (2 * p + (1 if h >= 4 else 0)) * 65 + (h % 4) * 16
                dst[p, pl.ds(h * 16, 16)] = spad[pl.ds(base, 16)]

    for b in range(4):
        read(start + b, b).start()

    @pl.loop(0, CT_PER_W // 4)
    def _(g):
        for b in range(4):
            ct = start + 4 * g + b
            read(ct, b).wait()

            @pl.when(g > 0)
            def _():
                write(ct - 4, b).wait()

            transpose(b, 64)
            write(ct, b).start()

            @pl.when(g < CT_PER_W // 4 - 1)
            def _():
                read(ct + 4, b).start()

    for b in range(4):
        write(start + CT_PER_W - 4 + b, b).wait()

    @pl.when(wid < 4)
    def _():
        ct = NW * CT_PER_W + wid
        pltpu.sync_copy(tab_hbm.at[:, pl.ds(ct * 128, 128)], in_v.at[0])
        transpose(0, 64)
        write(ct, 0).start()
        write(ct, 0).wait()

    @pl.when(wid == 31)
    def _():
        pltpu.sync_copy(tail_hbm, tr_v.at[1, pl.ds(0, CT_TAIL // 2)])
        pltpu.make_async_copy(
            tr_v.at[1, pl.ds(0, CT_TAIL // 2)],
            pack_hbm.at[pl.ds(NCT_FULL * 64, CT_TAIL // 2)], ws[1]).start()
        pltpu.make_async_copy(
            tr_v.at[1, pl.ds(0, CT_TAIL // 2)],
            pack_hbm.at[pl.ds(NCT_FULL * 64, CT_TAIL // 2)], ws[1]).wait()


def _lookup_body(pack_hbm, tok_hbm, out_hbm, tok_v, idx_v, gath_v, outb_v,
                 gpad, *sems):
    gs = sems[:4]
    os = sems[4:]
    wid = lax.axis_index("s") * NC + lax.axis_index("c")
    rows129 = [(h * 16 + _iota16()) * 129 for h in range(8)]

    def gather(s_i, b):
        return pltpu.make_async_copy(
            pack_hbm.at[idx_v.at[s_i]], gath_v.at[b], gs[b])

    def out_write(s, bct, b):
        return pltpu.make_async_copy(
            outb_v.at[b], out_hbm.at[s, :, bct], os[b])

    @pl.loop(0, UNITS_PER_W)
    def _(uu):
        u = wid * UNITS_PER_W + uu
        s0 = (u // 32) * 8
        bct = u % 32
        pltpu.sync_copy(tok_hbm.at[pl.ds(s0, 8), pl.ds(bct * 128, 128)],
                        tok_v)
        for s_i in range(8):
            for h in range(8):
                t = tok_v[s_i, pl.ds(h * 16, 16)]
                idx_v[s_i, pl.ds(h * 16, 16)] = lax.shift_right_logical(t, 1)

        for s_i in range(3):
            gather(s_i, s_i).start()
        for s_i in range(8):
            b = s_i % 4
            gather(s_i, b).wait()
            if s_i < 5:
                gather(s_i + 3, (s_i + 3) % 4).start()
            if s_i < 4:
                @pl.when(uu > 0)
                def _():
                    out_write(0, 0, b).wait()
            else:
                out_write(0, 0, b).wait()
            src = gath_v.at[b]
            dst = outb_v.at[b]

            @plsc.parallel_loop(0, 128, unroll=2)
            def _(bb):
                for h in range(8):
                    gpad[pl.ds(bb * 129 + h * 16, 16)] = (
                        src[bb, pl.ds(h * 16, 16)])

            bases = []
            for h in range(8):
                par = lax.bitwise_and(tok_v[s_i, pl.ds(h * 16, 16)], 1)
                bases.append(rows129[h] + par * 64)
            for fg in range(8):
                @plsc.parallel_loop(0, 8, unroll=2)
                def _(f8, fg=fg):
                    f = fg * 8 + f8
                    for h in range(8):
                        dst[fg, f8, pl.ds(h * 16, 16)] = plsc.load_gather(
                            gpad, [bases[h] + f])
            out_write(s0 + s_i, bct, b).start()

    for b in range(4):
        out_write(0, 0, b).wait()


@jax.jit
def kernel(tokens, emb_weight):
    tok_t = jnp.transpose(tokens).astype(jnp.int32)
    tab_t = jnp.transpose(emb_weight)
    tail2 = lax.slice(emb_weight, (NCT_FULL * 128, 0), (VOCAB, D_MODEL))
    tail2 = tail2.reshape(CT_TAIL // 2, 128)
    mesh = plsc.VectorSubcoreMesh(core_axis_name="c", subcore_axis_name="s")
    params = pltpu.CompilerParams(use_tc_tiling_on_sc=True,
                                  needs_layout_passes=False)

    pack = pl.kernel(
        _repack_body,
        out_type=jax.ShapeDtypeStruct((PAIRS, 128), jnp.float32),
        mesh=mesh,
        scratch_types=[
            pltpu.VMEM((4, 64, 128), jnp.float32),
            pltpu.VMEM((4, 64, 128), jnp.float32),
            pltpu.VMEM((64 * 65,), jnp.float32),
        ] + [pltpu.SemaphoreType.DMA] * 8,
        compiler_params=params,
    )(tab_t, tail2)

    out5 = pl.kernel(
        _lookup_body,
        out_type=jax.ShapeDtypeStruct((SEQ, 8, 32, 8, 128), jnp.float32),
        mesh=mesh,
        scratch_types=[
            pltpu.VMEM((8, 128), jnp.int32),
            pltpu.VMEM((8, 128), jnp.int32),
            pltpu.VMEM((4, 128, 128), jnp.float32),
            pltpu.VMEM((4, 8, 8, 128), jnp.float32),
            pltpu.VMEM((128 * 129,), jnp.float32),
        ] + [pltpu.SemaphoreType.DMA] * 8,
        compiler_params=params,
    )(pack, tok_t)

    out = jnp.transpose(out5, (2, 4, 0, 1, 3))
    return out.reshape(BATCH, SEQ, D_MODEL)
